# baseline (speedup 1.0000x reference)
import jax
import jax.numpy as jnp
from jax.experimental import pallas as pl
from jax.experimental.pallas import tpu as pltpu

B, C, T, IN_FEAT = 8, 64, 32, 75
GCN_HIDDEN, GCN_OUT, LSTM_HIDDEN, NUM_CLASSES = 128, 64, 128, 2
H_DIR = LSTM_HIDDEN // 2
EPS = 1e-06
E_KNN = 512 + C
G = 4

_F32 = jnp.float32
_BF = jnp.bfloat16
_HI = jax.lax.Precision.DEFAULT


def _dot(a, b, ca, cb, ba=(), bb=(), prec=_HI):
    return jax.lax.dot_general(
        a, b, ((ca, cb), (ba, bb)), precision=prec, preferred_element_type=_F32
    )


def _body(x4_ref, x2_ref, ei_ref, Wd1_ref, bd1_ref, Wd2_ref, bd2_ref,
          Wk1_ref, bk1_ref, Wk2_ref, bk2_ref, wc_ref, bc_ref,
          Wih0_ref, Whh0_ref, bg0_ref, Wih1_ref, Whh1_ref, bg1_ref,
          wt_ref, bt_ref, Wcls_ref, bcls_ref, out_ref, seq_ref, sk_ref):
    b = pl.program_id(0)
    x4 = x4_ref[...]
    xf = x2_ref[...]

    row = jax.lax.broadcasted_iota(jnp.int32, (C, C), 0)
    col = jax.lax.broadcasted_iota(jnp.int32, (C, C), 1)
    eye = (row == col).astype(_F32)

    mu = jnp.mean(xf, axis=-1, keepdims=True)
    xc = xf - mu
    std = jnp.sqrt(jnp.sum(xc * xc, axis=-1, keepdims=True) / xf.shape[-1])
    std = jnp.maximum(std, EPS)
    xn = xc / std
    corr = _dot(xn, xn, (2,), (2,), (0,), (0,)) / xf.shape[-1]
    m = jnp.where(jnp.abs(corr) >= 0.8, 1.0, 0.0)
    m = jnp.maximum(m, eye[None])
    Md = m + 2.0 * eye[None]
    deg = jnp.sum(Md, axis=2)
    dinv = jax.lax.rsqrt(deg)
    Sd = (dinv[:, :, None] * Md * dinv[:, None, :]).astype(_BF)

    @pl.when(b == 0)
    def _knn():
        src = ei_ref[0, :]
        dst = ei_ref[1, :]
        erow = jax.lax.broadcasted_iota(jnp.int32, (C, E_KNN), 0)
        dohT = (dst[None, :] == erow).astype(_F32)
        lane = jax.lax.broadcasted_iota(jnp.int32, (E_KNN, C), 1)
        soh = (src[:, None] == lane).astype(_F32)
        NT = _dot(dohT, soh, (1,), (0,))
        NeT = NT + eye
        degk = jnp.sum(NeT, axis=1)
        dk = jax.lax.rsqrt(degk)
        sk_ref[...] = (dk[:, None] * NeT * dk[None, :]).astype(_BF)

    SkT = jnp.broadcast_to(sk_ref[...][None], (G, C, C))

    def lin(h, W):
        return _dot(h, W, (3,), (0,)).astype(_BF)

    def prop(ST, xh):
        return _dot(ST, xh, (2,), (1,), (0,), (0,))

    def act(z, b_ref):
        return jax.nn.relu(z + b_ref[...]).astype(_BF)

    h1 = act(prop(Sd, lin(x4, Wd1_ref[...])), bd1_ref)
    hd = act(prop(Sd, lin(h1, Wd2_ref[...])), bd2_ref)
    k1 = act(prop(SkT, lin(x4, Wk1_ref[...])), bk1_ref)
    hk = act(prop(SkT, lin(k1, Wk2_ref[...])), bk2_ref)
    h = jnp.concatenate([hd, hk], axis=-1)

    scores = jnp.sum(h * wc_ref[...], axis=-1) + bc_ref[0, 0]
    watt = jax.nn.softmax(scores, axis=1)
    seq_ref[pl.ds(G * b, G)] = jnp.sum(watt[..., None] * h, axis=1)

    @pl.when(b == B // G - 1)
    def _tail():
        out_v = seq_ref[...]
        for Wih_ref, Whh_ref, bg_ref in ((Wih0_ref, Whh0_ref, bg0_ref),
                                         (Wih1_ref, Whh1_ref, bg1_ref)):
            Wih = Wih_ref[...]
            Whh = Whh_ref[...]
            bg = bg_ref[...]
            xcat = jnp.stack(
                [jnp.concatenate(
                    [out_v[:, t, :], out_v[:, T - 1 - t, :]], axis=-1)
                 for t in range(T)], axis=1)
            xp = _dot(xcat.astype(_BF), Wih, (2,), (0,)) + bg
            Hc = jnp.zeros((B, 2 * H_DIR), _F32)
            Cc = jnp.zeros((B, 2 * H_DIR), _F32)
            hs = []
            for t in range(T):
                g = xp[:, t, :] + _dot(Hc.astype(_BF), Whh, (1,), (0,))
                ig = jax.nn.sigmoid(g[:, 0:128])
                fg = jax.nn.sigmoid(g[:, 128:256])
                gg = jnp.tanh(g[:, 256:384])
                og = jax.nn.sigmoid(g[:, 384:512])
                Cc = fg * Cc + ig * gg
                Hc = og * jnp.tanh(Cc)
                hs.append(Hc)
            out_v = jnp.stack(
                [jnp.concatenate(
                    [hs[t][:, :H_DIR], hs[T - 1 - t][:, H_DIR:]], axis=-1)
                 for t in range(T)], axis=1)

        st = jnp.sum(out_v * wt_ref[...], axis=-1) + bt_ref[0, 0]
        wts = jax.nn.softmax(st, axis=1)
        ctx = jnp.sum(wts[..., None] * out_v, axis=1)
        out_ref[...] = _dot(ctx, Wcls_ref[...], (1,), (0,)) + bcls_ref[...]


def _comb_ih(Wf, Wr):
    din = Wf.shape[1]
    Wf_t = Wf.T.reshape(din, 4, H_DIR)
    Wr_t = Wr.T.reshape(din, 4, H_DIR)
    Wcmb = jnp.zeros((2 * din, 4, 2, H_DIR), _F32)
    Wcmb = Wcmb.at[:din, :, 0, :].set(Wf_t).at[din:, :, 1, :].set(Wr_t)
    return Wcmb.reshape(2 * din, 8 * H_DIR)


def _comb_hh(Wf, Wr):
    Wf_t = Wf.T.reshape(H_DIR, 4, H_DIR)
    Wr_t = Wr.T.reshape(H_DIR, 4, H_DIR)
    Wcmb = jnp.zeros((2 * H_DIR, 4, 2, H_DIR), _F32)
    Wcmb = Wcmb.at[:H_DIR, :, 0, :].set(Wf_t).at[H_DIR:, :, 1, :].set(Wr_t)
    return Wcmb.reshape(2 * H_DIR, 8 * H_DIR)


def _comb_b(bihf, bhhf, bihr, bhhr):
    bf = (bihf + bhhf).reshape(4, H_DIR)
    br = (bihr + bhhr).reshape(4, H_DIR)
    return jnp.stack([bf, br], axis=1).reshape(1, 8 * H_DIR)


def kernel(x, edge_index_knn, W_dyn1, b_dyn1, W_dyn2, b_dyn2, W_knn1, b_knn1,
           W_knn2, b_knn2, Wc, bc, Wt, bt, Wcls, bcls,
           Wih_l0f, Whh_l0f, bih_l0f, bhh_l0f, Wih_l0r, Whh_l0r, bih_l0r,
           bhh_l0r, Wih_l1f, Whh_l1f, bih_l1f, bhh_l1f, Wih_l1r, Whh_l1r,
           bih_l1r, bhh_l1r):
    x = x.astype(_F32)
    x2 = x.reshape(B, C, T * IN_FEAT)
    args = (
        x.astype(_BF), x2, edge_index_knn.astype(jnp.int32),
        W_dyn1.astype(_BF), b_dyn1.reshape(1, -1),
        W_dyn2.astype(_BF), b_dyn2.reshape(1, -1),
        W_knn1.astype(_BF), b_knn1.reshape(1, -1),
        W_knn2.astype(_BF), b_knn2.reshape(1, -1),
        Wc.reshape(1, -1), bc.reshape(1, 1),
        _comb_ih(Wih_l0f, Wih_l0r).astype(_BF),
        _comb_hh(Whh_l0f, Whh_l0r).astype(_BF),
        _comb_b(bih_l0f, bhh_l0f, bih_l0r, bhh_l0r),
        _comb_ih(Wih_l1f, Wih_l1r).astype(_BF),
        _comb_hh(Whh_l1f, Whh_l1r).astype(_BF),
        _comb_b(bih_l1f, bhh_l1f, bih_l1r, bhh_l1r),
        Wt.reshape(1, -1), bt.reshape(1, 1), Wcls, bcls.reshape(1, -1),
    )
    full = lambda s: pl.BlockSpec(s, lambda b: tuple(0 for _ in s))
    in_specs = [
        pl.BlockSpec((G, C, T, IN_FEAT), lambda b: (b, 0, 0, 0)),
        pl.BlockSpec((G, C, T * IN_FEAT), lambda b: (b, 0, 0)),
    ] + [full(a.shape) for a in args[2:]]
    return pl.pallas_call(
        _body,
        grid=(B // G,),
        in_specs=in_specs,
        out_specs=full((B, NUM_CLASSES)),
        out_shape=jax.ShapeDtypeStruct((B, NUM_CLASSES), _F32),
        scratch_shapes=[pltpu.VMEM((B, T, LSTM_HIDDEN), _F32),
                        pltpu.VMEM((C, C), _BF)],
    )(*args)


# speedup vs baseline: 1.0872x; 1.0872x over previous
import jax
import jax.numpy as jnp
from jax.experimental import pallas as pl
from jax.experimental.pallas import tpu as pltpu

B, C, T, IN_FEAT = 8, 64, 32, 75
GCN_HIDDEN, GCN_OUT, LSTM_HIDDEN, NUM_CLASSES = 128, 64, 128, 2
H_DIR = LSTM_HIDDEN // 2
EPS = 1e-06
E_KNN = 512 + C
G = 4

_F32 = jnp.float32
_BF = jnp.bfloat16
_HI = jax.lax.Precision.DEFAULT


def _dot(a, b, ca, cb, ba=(), bb=(), prec=_HI):
    return jax.lax.dot_general(
        a, b, ((ca, cb), (ba, bb)), precision=prec, preferred_element_type=_F32
    )


def _body(x4_ref, x2_ref, ei_ref, Wd1_ref, bd1_ref, Wd2_ref, bd2_ref,
          Wk1_ref, bk1_ref, Wk2_ref, bk2_ref, wc_ref, bc_ref,
          Wih0_ref, Whh0_ref, bg0_ref, Wih1_ref, Whh1_ref, bg1_ref,
          wt_ref, bt_ref, Wcls_ref, bcls_ref, out_ref, seq_ref, sk_ref):
    b = pl.program_id(0)
    x4 = x4_ref[...]
    xf = x2_ref[...]

    row = jax.lax.broadcasted_iota(jnp.int32, (C, C), 0)
    col = jax.lax.broadcasted_iota(jnp.int32, (C, C), 1)
    eye = (row == col).astype(_F32)

    n = xf.shape[-1]
    P = _dot(xf, xf, (2,), (2,), (0,), (0,))
    s = jnp.sum(xf, axis=-1) * (1.0 / jnp.sqrt(n))
    Pc = P - s[:, :, None] * s[:, None, :]
    var = jnp.sum(Pc * eye[None], axis=2, keepdims=True)
    std = jnp.maximum(jnp.sqrt(jnp.maximum(var / n, 0.0)), EPS)
    corr = Pc / (n * std * jnp.swapaxes(std, 1, 2))
    m = jnp.where(jnp.abs(corr) >= 0.8, 1.0, 0.0)
    m = jnp.maximum(m, eye[None])
    Md = m + 2.0 * eye[None]
    deg = jnp.sum(Md, axis=2)
    dinv = jax.lax.rsqrt(deg)
    Sd = dinv[:, :, None] * Md * dinv[:, None, :]

    @pl.when(b == 0)
    def _knn():
        src = ei_ref[0, :]
        dst = ei_ref[1, :]
        erow = jax.lax.broadcasted_iota(jnp.int32, (C, E_KNN), 0)
        dohT = (dst[None, :] == erow).astype(_F32)
        lane = jax.lax.broadcasted_iota(jnp.int32, (E_KNN, C), 1)
        soh = (src[:, None] == lane).astype(_F32)
        NT = _dot(dohT, soh, (1,), (0,))
        NeT = NT + eye
        degk = jnp.sum(NeT, axis=1)
        dk = jax.lax.rsqrt(degk)
        sk_ref[...] = dk[:, None] * NeT * dk[None, :]

    SkT = jnp.broadcast_to(sk_ref[...][None], (G, C, C))

    def lin(h, W):
        return _dot(h, W, (3,), (0,))

    def prop(ST, xh):
        return _dot(ST, xh, (2,), (1,), (0,), (0,))

    def act(z, b_ref):
        return jax.nn.relu(z + b_ref[...])

    h1 = act(prop(Sd, lin(x4, Wd1_ref[...])), bd1_ref)
    hd = act(prop(Sd, lin(h1, Wd2_ref[...])), bd2_ref)
    k1 = act(prop(SkT, lin(x4, Wk1_ref[...])), bk1_ref)
    hk = act(prop(SkT, lin(k1, Wk2_ref[...])), bk2_ref)
    h = jnp.concatenate([hd, hk], axis=-1)

    scores = jnp.sum(h * wc_ref[...], axis=-1) + bc_ref[0, 0]
    watt = jax.nn.softmax(scores, axis=1)
    seq_ref[pl.ds(G * b, G)] = jnp.sum(watt[..., None] * h, axis=1)

    @pl.when(b == B // G - 1)
    def _tail():
        out_v = seq_ref[...]
        for Wih_ref, Whh_ref, bg_ref in ((Wih0_ref, Whh0_ref, bg0_ref),
                                         (Wih1_ref, Whh1_ref, bg1_ref)):
            Wih = Wih_ref[...]
            Whh = Whh_ref[...]
            bg = bg_ref[...]
            xcat = jnp.stack(
                [jnp.concatenate(
                    [out_v[:, t, :], out_v[:, T - 1 - t, :]], axis=-1)
                 for t in range(T)], axis=1)
            xp = _dot(xcat, Wih, (2,), (0,)) + bg
            Hc = jnp.zeros((B, 2 * H_DIR), _F32)
            Cc = jnp.zeros((B, 2 * H_DIR), _F32)
            hs = []
            for t in range(T):
                g = xp[:, t, :] + _dot(Hc, Whh, (1,), (0,))
                ig = jax.nn.sigmoid(g[:, 0:128])
                fg = jax.nn.sigmoid(g[:, 128:256])
                gg = jnp.tanh(g[:, 256:384])
                og = jax.nn.sigmoid(g[:, 384:512])
                Cc = fg * Cc + ig * gg
                Hc = og * jnp.tanh(Cc)
                hs.append(Hc)
            out_v = jnp.stack(
                [jnp.concatenate(
                    [hs[t][:, :H_DIR], hs[T - 1 - t][:, H_DIR:]], axis=-1)
                 for t in range(T)], axis=1)

        st = jnp.sum(out_v * wt_ref[...], axis=-1) + bt_ref[0, 0]
        wts = jax.nn.softmax(st, axis=1)
        ctx = jnp.sum(wts[..., None] * out_v, axis=1)
        out_ref[...] = _dot(ctx, Wcls_ref[...], (1,), (0,)) + bcls_ref[...]


def _comb_ih(Wf, Wr):
    din = Wf.shape[1]
    Wf_t = Wf.T.reshape(din, 4, H_DIR)
    Wr_t = Wr.T.reshape(din, 4, H_DIR)
    Wcmb = jnp.zeros((2 * din, 4, 2, H_DIR), _F32)
    Wcmb = Wcmb.at[:din, :, 0, :].set(Wf_t).at[din:, :, 1, :].set(Wr_t)
    return Wcmb.reshape(2 * din, 8 * H_DIR)


def _comb_hh(Wf, Wr):
    Wf_t = Wf.T.reshape(H_DIR, 4, H_DIR)
    Wr_t = Wr.T.reshape(H_DIR, 4, H_DIR)
    Wcmb = jnp.zeros((2 * H_DIR, 4, 2, H_DIR), _F32)
    Wcmb = Wcmb.at[:H_DIR, :, 0, :].set(Wf_t).at[H_DIR:, :, 1, :].set(Wr_t)
    return Wcmb.reshape(2 * H_DIR, 8 * H_DIR)


def _comb_b(bihf, bhhf, bihr, bhhr):
    bf = (bihf + bhhf).reshape(4, H_DIR)
    br = (bihr + bhhr).reshape(4, H_DIR)
    return jnp.stack([bf, br], axis=1).reshape(1, 8 * H_DIR)


def kernel(x, edge_index_knn, W_dyn1, b_dyn1, W_dyn2, b_dyn2, W_knn1, b_knn1,
           W_knn2, b_knn2, Wc, bc, Wt, bt, Wcls, bcls,
           Wih_l0f, Whh_l0f, bih_l0f, bhh_l0f, Wih_l0r, Whh_l0r, bih_l0r,
           bhh_l0r, Wih_l1f, Whh_l1f, bih_l1f, bhh_l1f, Wih_l1r, Whh_l1r,
           bih_l1r, bhh_l1r):
    x = x.astype(_F32)
    x2 = x.reshape(B, C, T * IN_FEAT)
    args = (
        x, x2, edge_index_knn.astype(jnp.int32),
        W_dyn1, b_dyn1.reshape(1, -1), W_dyn2, b_dyn2.reshape(1, -1),
        W_knn1, b_knn1.reshape(1, -1), W_knn2, b_knn2.reshape(1, -1),
        Wc.reshape(1, -1), bc.reshape(1, 1),
        _comb_ih(Wih_l0f, Wih_l0r), _comb_hh(Whh_l0f, Whh_l0r),
        _comb_b(bih_l0f, bhh_l0f, bih_l0r, bhh_l0r),
        _comb_ih(Wih_l1f, Wih_l1r), _comb_hh(Whh_l1f, Whh_l1r),
        _comb_b(bih_l1f, bhh_l1f, bih_l1r, bhh_l1r),
        Wt.reshape(1, -1), bt.reshape(1, 1), Wcls, bcls.reshape(1, -1),
    )
    full = lambda s: pl.BlockSpec(s, lambda b: tuple(0 for _ in s))
    in_specs = [
        pl.BlockSpec((G, C, T, IN_FEAT), lambda b: (b, 0, 0, 0)),
        pl.BlockSpec((G, C, T * IN_FEAT), lambda b: (b, 0, 0)),
    ] + [full(a.shape) for a in args[2:]]
    return pl.pallas_call(
        _body,
        grid=(B // G,),
        in_specs=in_specs,
        out_specs=full((B, NUM_CLASSES)),
        out_shape=jax.ShapeDtypeStruct((B, NUM_CLASSES), _F32),
        scratch_shapes=[pltpu.VMEM((B, T, LSTM_HIDDEN), _F32),
                        pltpu.VMEM((C, C), _F32)],
    )(*args)
